# baseline (speedup 1.0000x reference)
import jax
import jax.numpy as jnp
from jax.experimental import pallas as pl
from jax.experimental.pallas import tpu as pltpu

FREEZE_RATIO = 0.9
ZBUF_ROWS = 1024
ALIGN = 8


def kernel(tokens):
    batch_size, seq_len, d_model = tokens.shape
    total = batch_size * seq_len
    target = int(total * FREEZE_RATIO)

    cut0 = (target // ALIGN) * ALIGN
    has_boundary = cut0 < total
    boundary_rows = min(ALIGN, total - cut0) if has_boundary else 0
    frozen_in_boundary = target - cut0
    live_start = cut0 + boundary_rows
    tail = total - live_start

    n_full = cut0 // ZBUF_ROWS
    rem = cut0 - n_full * ZBUF_ROWS
    n_zero = n_full + (1 if rem else 0)
    n_sem = n_zero + 3

    x = tokens.reshape(total, d_model)

    def body(x_hbm, out_hbm, count_ref, zbuf, bbuf, sem):
        count_ref[0] = target

        if tail:
            tail_copy = pltpu.make_async_copy(
                x_hbm.at[pl.ds(live_start, tail), :],
                out_hbm.at[pl.ds(live_start, tail), :],
                sem.at[n_zero],
            )
            tail_copy.start()
        if boundary_rows:
            b_in = pltpu.make_async_copy(
                x_hbm.at[pl.ds(cut0, boundary_rows), :],
                bbuf,
                sem.at[n_zero + 1],
            )
            b_in.start()

        zbuf[...] = jnp.zeros_like(zbuf)
        zcopies = []
        for k in range(n_full):
            c = pltpu.make_async_copy(
                zbuf,
                out_hbm.at[pl.ds(k * ZBUF_ROWS, ZBUF_ROWS), :],
                sem.at[k],
            )
            c.start()
            zcopies.append(c)
        if rem:
            c = pltpu.make_async_copy(
                zbuf.at[pl.ds(0, rem), :],
                out_hbm.at[pl.ds(n_full * ZBUF_ROWS, rem), :],
                sem.at[n_full],
            )
            c.start()
            zcopies.append(c)

        if boundary_rows:
            b_in.wait()
            if frozen_in_boundary:
                bbuf[0:frozen_in_boundary, :] = jnp.zeros(
                    (frozen_in_boundary, d_model), bbuf.dtype
                )
            b_out = pltpu.make_async_copy(
                bbuf,
                out_hbm.at[pl.ds(cut0, boundary_rows), :],
                sem.at[n_zero + 2],
            )
            b_out.start()

        for c in zcopies:
            c.wait()
        if tail:
            tail_copy.wait()
        if boundary_rows:
            b_out.wait()

    frozen_flat, count = pl.pallas_call(
        body,
        in_specs=[pl.BlockSpec(memory_space=pl.ANY)],
        out_specs=[
            pl.BlockSpec(memory_space=pl.ANY),
            pl.BlockSpec(memory_space=pltpu.SMEM),
        ],
        out_shape=[
            jax.ShapeDtypeStruct((total, d_model), tokens.dtype),
            jax.ShapeDtypeStruct((1,), jnp.int32),
        ],
        scratch_shapes=[
            pltpu.VMEM((ZBUF_ROWS, d_model), tokens.dtype),
            pltpu.VMEM((max(boundary_rows, 1), d_model), tokens.dtype),
            pltpu.SemaphoreType.DMA((n_sem,)),
        ],
    )(x)

    return (frozen_flat.reshape(batch_size, seq_len, d_model), count[0])


# speedup vs baseline: 7.7839x; 7.7839x over previous
import jax
import jax.numpy as jnp
from jax.experimental import pallas as pl
from jax.experimental.pallas import tpu as pltpu

FREEZE_RATIO = 0.9
ROWS_PER_BLOCK = 1024


def _freeze_body(target_smem, x_ref, out_ref, count_ref):
    i = pl.program_id(0)
    target = target_smem[0]
    row0 = i * ROWS_PER_BLOCK
    row_end = row0 + ROWS_PER_BLOCK

    @pl.when(i == 0)
    def _():
        count_ref[0] = target

    @pl.when(row_end <= target)
    def _():
        out_ref[...] = jnp.zeros_like(out_ref)

    @pl.when(row0 >= target)
    def _():
        out_ref[...] = x_ref[...]

    @pl.when(jnp.logical_and(row0 < target, row_end > target))
    def _():
        rows = row0 + jax.lax.broadcasted_iota(
            jnp.int32, out_ref.shape, 0
        )
        out_ref[...] = jnp.where(rows < target, 0.0, x_ref[...])


def kernel(tokens):
    batch_size, seq_len, d_model = tokens.shape
    total = batch_size * seq_len
    target = int(total * FREEZE_RATIO)
    assert total % ROWS_PER_BLOCK == 0
    num_blocks = total // ROWS_PER_BLOCK
    first_live = target // ROWS_PER_BLOCK

    x = tokens.reshape(total, d_model)

    frozen_flat, count = pl.pallas_call(
        _freeze_body,
        grid=(num_blocks,),
        in_specs=[
            pl.BlockSpec(memory_space=pltpu.SMEM),
            pl.BlockSpec(
                (ROWS_PER_BLOCK, d_model),
                lambda i: (jnp.maximum(i, first_live), 0),
            ),
        ],
        out_specs=[
            pl.BlockSpec((ROWS_PER_BLOCK, d_model), lambda i: (i, 0)),
            pl.BlockSpec(memory_space=pltpu.SMEM),
        ],
        out_shape=[
            jax.ShapeDtypeStruct((total, d_model), tokens.dtype),
            jax.ShapeDtypeStruct((1,), jnp.int32),
        ],
        compiler_params=pltpu.CompilerParams(
            dimension_semantics=("parallel",),
        ),
    )(jnp.full((1,), target, dtype=jnp.int32), x)

    frozen_tokens = frozen_flat.reshape(batch_size, seq_len, d_model)
    return (frozen_tokens, count[0])
